# baseline (speedup 1.0000x reference)
import functools

import jax
import jax.numpy as jnp
import numpy as np
from jax.experimental import pallas as pl
from jax.experimental.pallas import tpu as pltpu

_F32 = jnp.float32
_BIG = 3.0e38
_PREC = jax.lax.Precision.DEFAULT


def _dot(a, b):
    return jax.lax.dot_general(a, b, (((1,), (0,)), ((), ())),
                               preferred_element_type=_F32, precision=_PREC)


def _first_min_onehot(D, lane_iota, n):
    mn = jnp.min(D, axis=1, keepdims=True)
    idx = jnp.min(jnp.where(D == mn, lane_iota, n), axis=1, keepdims=True)
    oh = (lane_iota == idx).astype(_F32)
    return oh, mn, idx


def _fps_kernel(xs_ref, ys_ref, zs_ref, cx_ref, cy_ref, cz_ref, *, n, m):
    xs, ys, zs = xs_ref[...], ys_ref[...], zs_ref[...]
    x0, y0, z0 = xs[:, 0:1], ys[:, 0:1], zs[:, 0:1]
    iota = jax.lax.broadcasted_iota(jnp.int32, (1, n), 1)
    lane_m = jax.lax.broadcasted_iota(jnp.int32, (1, m), 1)
    d0 = (xs - x0) ** 2 + (ys - y0) ** 2 + (zs - z0) ** 2
    cx_ref[...] = jnp.broadcast_to(x0, cx_ref.shape)
    cy_ref[...] = jnp.broadcast_to(y0, cy_ref.shape)
    cz_ref[...] = jnp.broadcast_to(z0, cz_ref.shape)

    def body(i, dists):
        mx = jnp.max(dists, axis=1, keepdims=True)
        sel = jnp.min(jnp.where(dists == mx, iota, n), axis=1, keepdims=True)
        mask = (iota == sel).astype(_F32)
        xc = jnp.sum(xs * mask, axis=1, keepdims=True)
        yc = jnp.sum(ys * mask, axis=1, keepdims=True)
        zc = jnp.sum(zs * mask, axis=1, keepdims=True)
        dnew = (xs - xc) ** 2 + (ys - yc) ** 2 + (zs - zc) ** 2
        hit = lane_m == i
        cx_ref[...] = jnp.where(hit, xc, cx_ref[...])
        cy_ref[...] = jnp.where(hit, yc, cy_ref[...])
        cz_ref[...] = jnp.where(hit, zc, cz_ref[...])
        return jnp.minimum(dists, dnew)

    jax.lax.fori_loop(1, m, body, d0)


def _sa_kernel(featxyz_ref, cen_ref, xyzT_ref,
               w1_ref, b1_ref, w2_ref, b2_ref, w3_ref, b3_ref,
               fout_ref,
               *, n, m, k, cf):
    cen = cen_ref[...]
    sxyzT = xyzT_ref[...]
    D0 = ((cen[:, 0:1] - sxyzT[0:1, :]) ** 2
          + (cen[:, 1:2] - sxyzT[1:2, :]) ** 2)
    D0 = D0 + (cen[:, 2:3] - sxyzT[2:3, :]) ** 2

    lane_mn = jax.lax.broadcasted_iota(jnp.int32, (m, n), 1)
    cenpad = jnp.concatenate([jnp.zeros((m, cf), _F32), cen], axis=1)
    featxyz = featxyz_ref[...]
    w1, b1 = w1_ref[...], b1_ref[...]
    w2, b2 = w2_ref[...], b2_ref[...]
    w3, b3 = w3_ref[...], b3_ref[...]

    def nbr_body(_, carry):
        acc, D = carry
        oh, _, idx = _first_min_onehot(D, lane_mn, n)
        g = _dot(oh, featxyz) - cenpad
        a = jnp.maximum(_dot(g, w1) + b1, 0.0)
        a = jnp.maximum(_dot(a, w2) + b2, 0.0)
        a = jnp.maximum(_dot(a, w3) + b3, 0.0)
        return jnp.maximum(acc, a), D + oh * _BIG

    cout = w3.shape[1]
    fout_ref[...] = jax.lax.fori_loop(
        0, k, nbr_body, (jnp.zeros((m, cout), _F32), D0))[0]


def _sa(layers, feat, xyz, m, k):
    G, n, cf = feat.shape
    (w1, b1), (w2, b2), (w3, b3) = layers
    cout = w3.shape[1]
    featxyz = jnp.concatenate([feat, xyz], axis=2)
    xyzT = jnp.transpose(xyz, (0, 2, 1))
    cx, cy, cz = pl.pallas_call(
        functools.partial(_fps_kernel, n=n, m=m),
        out_shape=(jax.ShapeDtypeStruct((G, m), _F32),) * 3,
    )(xyzT[:, 0], xyzT[:, 1], xyzT[:, 2])
    cen = jnp.stack([cx, cy, cz], axis=2)
    fn = pl.pallas_call(
        functools.partial(_sa_kernel, n=n, m=m, k=k, cf=cf),
        out_shape=jax.ShapeDtypeStruct((m, cout), _F32),
    )
    fout = jax.vmap(fn, in_axes=(0, 0, 0) + (None,) * 6)(
        featxyz, cen, xyzT,
        w1, b1.reshape(1, -1), w2, b2.reshape(1, -1), w3, b3.reshape(1, -1))
    return fout, cen


def _lpt_kernel(fcur_ref, fsrc_ref, qxyz_ref, sxyzT_ref,
                wq_ref, wk_ref, wv_ref, out_ref, s_ref, i_ref,
                *, m, n, k, c):
    qxyz = qxyz_ref[...]
    sxyzT = sxyzT_ref[...]
    D0 = ((qxyz[:, 0:1] - sxyzT[0:1, :]) ** 2
          + (qxyz[:, 1:2] - sxyzT[1:2, :]) ** 2)
    D0 = D0 + (qxyz[:, 2:3] - sxyzT[2:3, :]) ** 2
    lane_mn = jax.lax.broadcasted_iota(jnp.int32, (m, n), 1)

    q = _dot(fcur_ref[...], wq_ref[...])
    K = _dot(fsrc_ref[...], wk_ref[...])
    V = _dot(fsrc_ref[...], wv_ref[...])
    S = jax.lax.dot_general(q, K, (((1,), (1,)), ((), ())),
                            preferred_element_type=_F32,
                            precision=_PREC)

    def sel_body(j, D):
        oh, _, idx = _first_min_onehot(D, lane_mn, n)
        s_ref[j] = jnp.sum(S * oh, axis=1, keepdims=True)
        i_ref[j] = idx
        return D + oh * _BIG

    jax.lax.fori_loop(0, k, sel_body, D0)

    s = s_ref[...] / np.sqrt(c)
    e = jnp.exp(s - jnp.max(s, axis=0, keepdims=True))
    s_ref[...] = e / jnp.sum(e, axis=0, keepdims=True)

    def mix_body(j, A):
        return A + s_ref[j] * (lane_mn == i_ref[j]).astype(_F32)

    A = jax.lax.fori_loop(0, k, mix_body, jnp.zeros((m, n), _F32))
    out_ref[...] = _dot(A, V)


def _lpt(p, f_cur, f_src, xyz_cur, xyz_src, k):
    B, m, c = f_cur.shape
    n = f_src.shape[1]
    sxyzT = jnp.transpose(xyz_src, (0, 2, 1))
    fn = pl.pallas_call(
        functools.partial(_lpt_kernel, m=m, n=n, k=k, c=c),
        out_shape=jax.ShapeDtypeStruct((m, c), _F32),
        scratch_shapes=[pltpu.VMEM((k, m, 1), _F32),
                        pltpu.VMEM((k, m, 1), jnp.int32)],
    )
    return jax.vmap(fn, in_axes=(0, 0, 0, 0, None, None, None))(
        f_cur, f_src, xyz_cur, sxyzT, p['Wq'], p['Wk'], p['Wv'])


def _lstm3_kernel(*refs):
    for lvl in range(3):
        fb_ref, ff_ref, h_ref, c_ref = refs[4 * lvl:4 * lvl + 4]
        wx_ref, wh_ref, b_ref = refs[12 + 3 * lvl:15 + 3 * lvl]
        hout_ref, cout_ref = refs[21 + 2 * lvl:23 + 2 * lvl]
        hdim = h_ref.shape[1]
        x = jnp.concatenate([fb_ref[...], ff_ref[...]], axis=1)
        g = _dot(x, wx_ref[...]) + _dot(h_ref[...], wh_ref[...]) + b_ref[...]
        i = jax.nn.sigmoid(g[:, 0:hdim])
        f = jax.nn.sigmoid(g[:, hdim:2 * hdim])
        gg = jnp.tanh(g[:, 2 * hdim:3 * hdim])
        o = jax.nn.sigmoid(g[:, 3 * hdim:4 * hdim])
        cn = f * c_ref[...] + i * gg
        hout_ref[...] = o * jnp.tanh(cn)
        cout_ref[...] = cn


def _lstm3(p, st, fbff):
    H1, C1, H2, C2, H3, C3 = st
    fb1, ff1, fb2, ff2, fb3, ff3 = fbff
    B = H1.shape[0]
    shapes = tuple(jax.ShapeDtypeStruct(h.shape[1:], _F32)
                   for h in (H1, H1, H2, H2, H3, H3))
    fn = pl.pallas_call(_lstm3_kernel, out_shape=shapes)
    ws = []
    for name in ('lstm1', 'lstm2', 'lstm3'):
        ws += [p[name]['Wx'], p[name]['Wh'], p[name]['b'].reshape(1, -1)]
    return jax.vmap(fn, in_axes=(0,) * 12 + (None,) * 9)(
        fb1, ff1, H1, C1, fb2, ff2, H2, C2, fb3, ff3, H3, C3, *ws)


def _interp(xc, posf, poscT, m, n, k):
    D0 = ((posf[:, 0:1] - poscT[0:1, :]) ** 2
          + (posf[:, 1:2] - poscT[1:2, :]) ** 2)
    D0 = D0 + (posf[:, 2:3] - poscT[2:3, :]) ** 2
    lane_mn = jax.lax.broadcasted_iota(jnp.int32, (m, n), 1)

    def body(_, carry):
        W, wsum, D = carry
        oh, mn, idx = _first_min_onehot(D, lane_mn, n)
        w = 1.0 / (mn + 1e-2)
        return W + w * oh, wsum + w, D + oh * _BIG

    W, wsum, _ = jax.lax.fori_loop(
        0, k, body, (jnp.zeros((m, n), _F32), jnp.zeros((m, 1), _F32), D0))
    return _dot(W, xc) / wsum


def _fp_kernel(xc_ref, posf_ref, poscT_ref, xskip_ref,
               w1_ref, b1_ref, w2_ref, b2_ref, out_ref, *, m, n, k):
    interp = _interp(xc_ref[...], posf_ref[...], poscT_ref[...], m, n, k)
    h = jnp.concatenate([interp, xskip_ref[...]], axis=1)
    h = jnp.maximum(_dot(h, w1_ref[...]) + b1_ref[...], 0.0)
    h = jnp.maximum(_dot(h, w2_ref[...]) + b2_ref[...], 0.0)
    out_ref[...] = h


def _fp(layers, x_c, pos_c, x_skip, pos_f, k):
    B, m, _ = pos_f.shape
    n = pos_c.shape[1]
    (w1, b1), (w2, b2) = layers
    poscT = jnp.transpose(pos_c, (0, 2, 1))
    fn = pl.pallas_call(
        functools.partial(_fp_kernel, m=m, n=n, k=k),
        out_shape=jax.ShapeDtypeStruct((m, w2.shape[1]), _F32),
    )
    return jax.vmap(fn, in_axes=(0, 0, 0, 0, None, None, None, None))(
        x_c, pos_f, poscT, x_skip, w1, b1.reshape(1, -1), w2, b2.reshape(1, -1))


def _fpns_cls_kernel(xc_ref, posf_ref, poscT_ref,
                     w1_ref, b1_ref, w2_ref, b2_ref,
                     c1_ref, c2_ref, c3_ref, c4_ref, out_ref,
                     *, m, n, k):
    interp = _interp(xc_ref[...], posf_ref[...], poscT_ref[...], m, n, k)
    h = jnp.maximum(_dot(interp, w1_ref[...]) + b1_ref[...], 0.0)
    h = jnp.maximum(_dot(h, w2_ref[...]) + b2_ref[...], 0.0)
    h = _dot(h, c1_ref[...])
    h = _dot(h, c2_ref[...])
    h = _dot(h, c3_ref[...])
    h = _dot(h, c4_ref[...])
    out_ref[...] = posf_ref[...] + h


def _fpns_cls(layers, cls, x_c, pos_c, pos_f, k):
    B, m, _ = pos_f.shape
    n = pos_c.shape[1]
    (w1, b1), (w2, b2) = layers
    c1, c2, c3, c4 = cls
    poscT = jnp.transpose(pos_c, (0, 2, 1))
    fn = pl.pallas_call(
        functools.partial(_fpns_cls_kernel, m=m, n=n, k=k),
        out_shape=jax.ShapeDtypeStruct((m, 3), _F32),
    )
    return jax.vmap(fn, in_axes=(0, 0, 0) + (None,) * 8)(
        x_c, pos_f, poscT, w1, b1.reshape(1, -1), w2, b2.reshape(1, -1),
        c1, c2, c3, c4)


def kernel(input_xyz, num_pred, params):
    p = params
    T, B, _, N = input_xyz.shape
    frames = jnp.transpose(input_xyz, (0, 1, 3, 2))
    N1, N2, N3 = N // 16, N // 32, N // 64

    def encode(fr):
        f1, x1 = _sa(p['sa1'], fr, fr, N1, 32)
        f2, x2 = _sa(p['sa2'], f1, x1, N2, 16)
        f3, x3 = _sa(p['sa3'], f2, x2, N3, 8)
        return (f1, x1, f2, x2, f3, x3)

    e_all = encode(frames.reshape(T * B, N, 3))
    encs = [tuple(a.reshape((T, B) + a.shape[1:])[t] for a in e_all)
            for t in range(T)]

    st = (jnp.zeros((B, N1, 128), _F32), jnp.zeros((B, N1, 128), _F32),
          jnp.zeros((B, N2, 256), _F32), jnp.zeros((B, N2, 256), _F32),
          jnp.zeros((B, N3, 512), _F32), jnp.zeros((B, N3, 512), _F32))

    def lpt_all(pairs, p_l, k_l, fi, pi):
        f_cur = jnp.concatenate([c[fi] for c, _ in pairs], axis=0)
        f_src = jnp.concatenate([s[fi] for _, s in pairs], axis=0)
        x_cur = jnp.concatenate([c[pi] for c, _ in pairs], axis=0)
        x_src = jnp.concatenate([s[pi] for _, s in pairs], axis=0)
        out = _lpt(p_l, f_cur, f_src, x_cur, x_src, k_l)
        return out.reshape((len(pairs), B) + out.shape[1:])

    def lpt_levels(pairs):
        a1 = lpt_all(pairs, p['gat1'], 16, 0, 1)
        a2 = lpt_all(pairs, p['gat2'], 16, 2, 3)
        a3 = lpt_all(pairs, p['gat3'], 8, 4, 5)
        return a1, a2, a3

    pairs = []
    for t in range(T):
        prev = encs[t - 1] if t > 0 else encs[0]
        nxt = encs[t + 1] if t < T - 1 else encs[t]
        pairs += [(encs[t], prev), (encs[t], nxt)]
    a1, a2, a3 = lpt_levels(pairs)
    for t in range(T):
        st = _lstm3(p, st, (a1[2 * t], a1[2 * t + 1], a2[2 * t],
                            a2[2 * t + 1], a3[2 * t], a3[2 * t + 1]))

    def decode(st, e, fine_xyz):
        H1, _, H2, _, H3, _ = st
        x2 = _fp(p['fp32'], H3, e[5], H2, e[3], 8)
        x1 = _fp(p['fp21'], x2, e[3], H1, e[1], 16)
        return _fpns_cls(p['fp10'], p['cls'], x1, e[1], fine_xyz, 32)

    num_steps = 2
    pc_next = decode(st, encs[-1], frames[-1])
    preds = [pc_next]
    for _ in range(1, num_steps):
        e_new = encode(pc_next)
        b1, b2, b3 = lpt_levels([(e_new, encs[-1]), (e_new, e_new)])
        st = _lstm3(p, st, (b1[0], b1[1], b2[0], b2[1], b3[0], b3[1]))
        encs.append(e_new)
        pc_next = decode(st, e_new, pc_next)
        preds.append(pc_next)
    return jnp.stack(preds)


# speedup vs baseline: 1.1143x; 1.1143x over previous
import functools

import jax
import jax.numpy as jnp
import numpy as np
from jax.experimental import pallas as pl
from jax.experimental.pallas import tpu as pltpu

_F32 = jnp.float32
_BIG = 3.0e38
_PREC = jax.lax.Precision.DEFAULT


def _dot(a, b):
    return jax.lax.dot_general(a, b, (((1,), (0,)), ((), ())),
                               preferred_element_type=_F32, precision=_PREC)


def _first_min_onehot(D, lane_iota, n):
    mn = jnp.min(D, axis=1, keepdims=True)
    idx = jnp.min(jnp.where(D == mn, lane_iota, n), axis=1, keepdims=True)
    oh = (lane_iota == idx).astype(_F32)
    return oh, mn, idx


def _fps_kernel(xs_ref, ys_ref, zs_ref, cx_ref, cy_ref, cz_ref, *, n, m):
    xs, ys, zs = xs_ref[...], ys_ref[...], zs_ref[...]
    x0, y0, z0 = xs[:, 0:1], ys[:, 0:1], zs[:, 0:1]
    iota = jax.lax.broadcasted_iota(jnp.int32, (1, n), 1)
    lane_m = jax.lax.broadcasted_iota(jnp.int32, (1, m), 1)
    d0 = (xs - x0) ** 2 + (ys - y0) ** 2 + (zs - z0) ** 2
    cx_ref[...] = jnp.broadcast_to(x0, cx_ref.shape)
    cy_ref[...] = jnp.broadcast_to(y0, cy_ref.shape)
    cz_ref[...] = jnp.broadcast_to(z0, cz_ref.shape)

    def body(i, dists):
        mx = jnp.max(dists, axis=1, keepdims=True)
        sel = jnp.min(jnp.where(dists == mx, iota, n), axis=1, keepdims=True)
        mask = (iota == sel).astype(_F32)
        xc = jnp.sum(xs * mask, axis=1, keepdims=True)
        yc = jnp.sum(ys * mask, axis=1, keepdims=True)
        zc = jnp.sum(zs * mask, axis=1, keepdims=True)
        dnew = (xs - xc) ** 2 + (ys - yc) ** 2 + (zs - zc) ** 2
        hit = lane_m == i
        cx_ref[...] = jnp.where(hit, xc, cx_ref[...])
        cy_ref[...] = jnp.where(hit, yc, cy_ref[...])
        cz_ref[...] = jnp.where(hit, zc, cz_ref[...])
        return jnp.minimum(dists, dnew)

    jax.lax.fori_loop(1, m, body, d0)


def _sa_kernel(featxyz_ref, cen_ref, xyzT_ref,
               w1_ref, b1_ref, w2_ref, b2_ref, w3_ref, b3_ref,
               fout_ref, d_ref,
               *, n, m, k, cf):
    cen = cen_ref[...]
    sxyzT = xyzT_ref[...]
    D0 = ((cen[:, 0:1] - sxyzT[0:1, :]) ** 2
          + (cen[:, 1:2] - sxyzT[1:2, :]) ** 2)
    d_ref[...] = D0 + (cen[:, 2:3] - sxyzT[2:3, :]) ** 2

    lane_mn = jax.lax.broadcasted_iota(jnp.int32, (m, n), 1)
    cenpad = jnp.concatenate([jnp.zeros((m, cf), _F32), cen], axis=1)
    featxyz = featxyz_ref[...]
    w1, b1 = w1_ref[...], b1_ref[...]
    w2, b2 = w2_ref[...], b2_ref[...]
    w3, b3 = w3_ref[...], b3_ref[...]

    def nbr_body(_, acc):
        D = d_ref[...]
        oh, _, idx = _first_min_onehot(D, lane_mn, n)
        d_ref[...] = D + oh * _BIG
        g = _dot(oh, featxyz) - cenpad
        a = jnp.maximum(_dot(g, w1) + b1, 0.0)
        a = jnp.maximum(_dot(a, w2) + b2, 0.0)
        a = jnp.maximum(_dot(a, w3) + b3, 0.0)
        return jnp.maximum(acc, a)

    cout = w3.shape[1]
    fout_ref[...] = jax.lax.fori_loop(
        0, k, nbr_body, jnp.zeros((m, cout), _F32))


def _sa(layers, feat, xyz, m, k):
    G, n, cf = feat.shape
    (w1, b1), (w2, b2), (w3, b3) = layers
    cout = w3.shape[1]
    featxyz = jnp.concatenate([feat, xyz], axis=2)
    xyzT = jnp.transpose(xyz, (0, 2, 1))
    cx, cy, cz = pl.pallas_call(
        functools.partial(_fps_kernel, n=n, m=m),
        out_shape=(jax.ShapeDtypeStruct((G, m), _F32),) * 3,
    )(xyzT[:, 0], xyzT[:, 1], xyzT[:, 2])
    cen = jnp.stack([cx, cy, cz], axis=2)
    fn = pl.pallas_call(
        functools.partial(_sa_kernel, n=n, m=m, k=k, cf=cf),
        out_shape=jax.ShapeDtypeStruct((m, cout), _F32),
        scratch_shapes=[pltpu.VMEM((m, n), _F32)],
    )
    fout = jax.vmap(fn, in_axes=(0, 0, 0) + (None,) * 6)(
        featxyz, cen, xyzT,
        w1, b1.reshape(1, -1), w2, b2.reshape(1, -1), w3, b3.reshape(1, -1))
    return fout, cen


def _lpt_kernel(fcur_ref, fsrc_ref, qxyz_ref, sxyzT_ref,
                wq_ref, wk_ref, wv_ref, out_ref, d_ref, s_ref, i_ref,
                *, m, n, k, c):
    qxyz = qxyz_ref[...]
    sxyzT = sxyzT_ref[...]
    D0 = ((qxyz[:, 0:1] - sxyzT[0:1, :]) ** 2
          + (qxyz[:, 1:2] - sxyzT[1:2, :]) ** 2)
    d_ref[...] = D0 + (qxyz[:, 2:3] - sxyzT[2:3, :]) ** 2
    lane_mn = jax.lax.broadcasted_iota(jnp.int32, (m, n), 1)

    q = _dot(fcur_ref[...], wq_ref[...])
    K = _dot(fsrc_ref[...], wk_ref[...])
    V = _dot(fsrc_ref[...], wv_ref[...])
    S = jax.lax.dot_general(q, K, (((1,), (1,)), ((), ())),
                            preferred_element_type=_F32,
                            precision=_PREC)

    def sel_body(j, _):
        D = d_ref[...]
        oh, _, idx = _first_min_onehot(D, lane_mn, n)
        d_ref[...] = D + oh * _BIG
        s_ref[j] = jnp.sum(S * oh, axis=1, keepdims=True)
        i_ref[j] = idx
        return 0

    jax.lax.fori_loop(0, k, sel_body, 0)

    s = s_ref[...] / np.sqrt(c)
    e = jnp.exp(s - jnp.max(s, axis=0, keepdims=True))
    s_ref[...] = e / jnp.sum(e, axis=0, keepdims=True)

    def mix_body(j, A):
        return A + s_ref[j] * (lane_mn == i_ref[j]).astype(_F32)

    A = jax.lax.fori_loop(0, k, mix_body, jnp.zeros((m, n), _F32))
    out_ref[...] = _dot(A, V)


def _lpt(p, f_cur, f_src, xyz_cur, xyz_src, k):
    B, m, c = f_cur.shape
    n = f_src.shape[1]
    sxyzT = jnp.transpose(xyz_src, (0, 2, 1))
    fn = pl.pallas_call(
        functools.partial(_lpt_kernel, m=m, n=n, k=k, c=c),
        out_shape=jax.ShapeDtypeStruct((m, c), _F32),
        scratch_shapes=[pltpu.VMEM((m, n), _F32),
                        pltpu.VMEM((k, m, 1), _F32),
                        pltpu.VMEM((k, m, 1), jnp.int32)],
    )
    return jax.vmap(fn, in_axes=(0, 0, 0, 0, None, None, None))(
        f_cur, f_src, xyz_cur, sxyzT, p['Wq'], p['Wk'], p['Wv'])


def _lstm3_kernel(*refs):
    for lvl in range(3):
        fb_ref, ff_ref, h_ref, c_ref = refs[4 * lvl:4 * lvl + 4]
        wx_ref, wh_ref, b_ref = refs[12 + 3 * lvl:15 + 3 * lvl]
        hout_ref, cout_ref = refs[21 + 2 * lvl:23 + 2 * lvl]
        hdim = h_ref.shape[1]
        x = jnp.concatenate([fb_ref[...], ff_ref[...]], axis=1)
        g = _dot(x, wx_ref[...]) + _dot(h_ref[...], wh_ref[...]) + b_ref[...]
        i = jax.nn.sigmoid(g[:, 0:hdim])
        f = jax.nn.sigmoid(g[:, hdim:2 * hdim])
        gg = jnp.tanh(g[:, 2 * hdim:3 * hdim])
        o = jax.nn.sigmoid(g[:, 3 * hdim:4 * hdim])
        cn = f * c_ref[...] + i * gg
        hout_ref[...] = o * jnp.tanh(cn)
        cout_ref[...] = cn


def _lstm3(p, st, fbff):
    H1, C1, H2, C2, H3, C3 = st
    fb1, ff1, fb2, ff2, fb3, ff3 = fbff
    B = H1.shape[0]
    shapes = tuple(jax.ShapeDtypeStruct(h.shape[1:], _F32)
                   for h in (H1, H1, H2, H2, H3, H3))
    fn = pl.pallas_call(_lstm3_kernel, out_shape=shapes)
    ws = []
    for name in ('lstm1', 'lstm2', 'lstm3'):
        ws += [p[name]['Wx'], p[name]['Wh'], p[name]['b'].reshape(1, -1)]
    return jax.vmap(fn, in_axes=(0,) * 12 + (None,) * 9)(
        fb1, ff1, H1, C1, fb2, ff2, H2, C2, fb3, ff3, H3, C3, *ws)


def _interp(xc, posf, poscT, d_ref, m, n, k):
    D0 = ((posf[:, 0:1] - poscT[0:1, :]) ** 2
          + (posf[:, 1:2] - poscT[1:2, :]) ** 2)
    d_ref[...] = D0 + (posf[:, 2:3] - poscT[2:3, :]) ** 2
    lane_mn = jax.lax.broadcasted_iota(jnp.int32, (m, n), 1)

    def body(_, carry):
        W, wsum = carry
        D = d_ref[...]
        oh, mn, idx = _first_min_onehot(D, lane_mn, n)
        d_ref[...] = D + oh * _BIG
        w = 1.0 / (mn + 1e-2)
        return W + w * oh, wsum + w

    W, wsum = jax.lax.fori_loop(
        0, k, body, (jnp.zeros((m, n), _F32), jnp.zeros((m, 1), _F32)))
    return _dot(W, xc) / wsum


def _fp_kernel(xc_ref, posf_ref, poscT_ref, xskip_ref,
               w1_ref, b1_ref, w2_ref, b2_ref, out_ref, d_ref, *, m, n, k):
    interp = _interp(xc_ref[...], posf_ref[...], poscT_ref[...], d_ref, m, n, k)
    h = jnp.concatenate([interp, xskip_ref[...]], axis=1)
    h = jnp.maximum(_dot(h, w1_ref[...]) + b1_ref[...], 0.0)
    h = jnp.maximum(_dot(h, w2_ref[...]) + b2_ref[...], 0.0)
    out_ref[...] = h


def _fp(layers, x_c, pos_c, x_skip, pos_f, k):
    B, m, _ = pos_f.shape
    n = pos_c.shape[1]
    (w1, b1), (w2, b2) = layers
    poscT = jnp.transpose(pos_c, (0, 2, 1))
    fn = pl.pallas_call(
        functools.partial(_fp_kernel, m=m, n=n, k=k),
        out_shape=jax.ShapeDtypeStruct((m, w2.shape[1]), _F32),
        scratch_shapes=[pltpu.VMEM((m, n), _F32)],
    )
    return jax.vmap(fn, in_axes=(0, 0, 0, 0, None, None, None, None))(
        x_c, pos_f, poscT, x_skip, w1, b1.reshape(1, -1), w2, b2.reshape(1, -1))


def _fpns_cls_kernel(xc_ref, posf_ref, poscT_ref,
                     w1_ref, b1_ref, w2_ref, b2_ref,
                     c1_ref, c2_ref, c3_ref, c4_ref, out_ref, d_ref,
                     *, m, n, k):
    interp = _interp(xc_ref[...], posf_ref[...], poscT_ref[...], d_ref, m, n, k)
    h = jnp.maximum(_dot(interp, w1_ref[...]) + b1_ref[...], 0.0)
    h = jnp.maximum(_dot(h, w2_ref[...]) + b2_ref[...], 0.0)
    h = _dot(h, c1_ref[...])
    h = _dot(h, c2_ref[...])
    h = _dot(h, c3_ref[...])
    h = _dot(h, c4_ref[...])
    out_ref[...] = posf_ref[...] + h


def _fpns_cls(layers, cls, x_c, pos_c, pos_f, k):
    B, m, _ = pos_f.shape
    n = pos_c.shape[1]
    (w1, b1), (w2, b2) = layers
    c1, c2, c3, c4 = cls
    poscT = jnp.transpose(pos_c, (0, 2, 1))
    fn = pl.pallas_call(
        functools.partial(_fpns_cls_kernel, m=m, n=n, k=k),
        out_shape=jax.ShapeDtypeStruct((m, 3), _F32),
        scratch_shapes=[pltpu.VMEM((m, n), _F32)],
    )
    return jax.vmap(fn, in_axes=(0, 0, 0) + (None,) * 8)(
        x_c, pos_f, poscT, w1, b1.reshape(1, -1), w2, b2.reshape(1, -1),
        c1, c2, c3, c4)


def kernel(input_xyz, num_pred, params):
    p = params
    T, B, _, N = input_xyz.shape
    frames = jnp.transpose(input_xyz, (0, 1, 3, 2))
    N1, N2, N3 = N // 16, N // 32, N // 64

    def encode(fr):
        f1, x1 = _sa(p['sa1'], fr, fr, N1, 32)
        f2, x2 = _sa(p['sa2'], f1, x1, N2, 16)
        f3, x3 = _sa(p['sa3'], f2, x2, N3, 8)
        return (f1, x1, f2, x2, f3, x3)

    e_all = encode(frames.reshape(T * B, N, 3))
    encs = [tuple(a.reshape((T, B) + a.shape[1:])[t] for a in e_all)
            for t in range(T)]

    st = (jnp.zeros((B, N1, 128), _F32), jnp.zeros((B, N1, 128), _F32),
          jnp.zeros((B, N2, 256), _F32), jnp.zeros((B, N2, 256), _F32),
          jnp.zeros((B, N3, 512), _F32), jnp.zeros((B, N3, 512), _F32))

    def lpt_all(pairs, p_l, k_l, fi, pi):
        f_cur = jnp.concatenate([c[fi] for c, _ in pairs], axis=0)
        f_src = jnp.concatenate([s[fi] for _, s in pairs], axis=0)
        x_cur = jnp.concatenate([c[pi] for c, _ in pairs], axis=0)
        x_src = jnp.concatenate([s[pi] for _, s in pairs], axis=0)
        out = _lpt(p_l, f_cur, f_src, x_cur, x_src, k_l)
        return out.reshape((len(pairs), B) + out.shape[1:])

    def lpt_levels(pairs):
        a1 = lpt_all(pairs, p['gat1'], 16, 0, 1)
        a2 = lpt_all(pairs, p['gat2'], 16, 2, 3)
        a3 = lpt_all(pairs, p['gat3'], 8, 4, 5)
        return a1, a2, a3

    pairs = []
    for t in range(T):
        prev = encs[t - 1] if t > 0 else encs[0]
        nxt = encs[t + 1] if t < T - 1 else encs[t]
        pairs += [(encs[t], prev), (encs[t], nxt)]
    a1, a2, a3 = lpt_levels(pairs)
    for t in range(T):
        st = _lstm3(p, st, (a1[2 * t], a1[2 * t + 1], a2[2 * t],
                            a2[2 * t + 1], a3[2 * t], a3[2 * t + 1]))

    def decode(st, e, fine_xyz):
        H1, _, H2, _, H3, _ = st
        x2 = _fp(p['fp32'], H3, e[5], H2, e[3], 8)
        x1 = _fp(p['fp21'], x2, e[3], H1, e[1], 16)
        return _fpns_cls(p['fp10'], p['cls'], x1, e[1], fine_xyz, 32)

    num_steps = 2
    pc_next = decode(st, encs[-1], frames[-1])
    preds = [pc_next]
    for _ in range(1, num_steps):
        e_new = encode(pc_next)
        b1, b2, b3 = lpt_levels([(e_new, encs[-1]), (e_new, e_new)])
        st = _lstm3(p, st, (b1[0], b1[1], b2[0], b2[1], b3[0], b3[1]))
        encs.append(e_new)
        pc_next = decode(st, e_new, pc_next)
        preds.append(pc_next)
    return jnp.stack(preds)


# speedup vs baseline: 1.1575x; 1.0388x over previous
import functools

import jax
import jax.numpy as jnp
import numpy as np
from jax import lax
from jax.experimental import pallas as pl
from jax.experimental.pallas import tpu as pltpu
from jax.experimental.pallas import tpu_sc as plsc

_F32 = jnp.float32
_BIG = 3.0e38
_PREC = jax.lax.Precision.DEFAULT


def _dot(a, b):
    return jax.lax.dot_general(a, b, (((1,), (0,)), ((), ())),
                               preferred_element_type=_F32, precision=_PREC)


def _first_min_onehot(D, lane_iota, n):
    mn = jnp.min(D, axis=1, keepdims=True)
    idx = jnp.min(jnp.where(D == mn, lane_iota, n), axis=1, keepdims=True)
    oh = (lane_iota == idx).astype(_F32)
    return oh, mn, idx


def _fps_kernel(xs_ref, ys_ref, zs_ref, cx_ref, cy_ref, cz_ref, *, n, m):
    xs, ys, zs = xs_ref[...], ys_ref[...], zs_ref[...]
    x0, y0, z0 = xs[:, 0:1], ys[:, 0:1], zs[:, 0:1]
    iota = jax.lax.broadcasted_iota(jnp.int32, (1, n), 1)
    lane_m = jax.lax.broadcasted_iota(jnp.int32, (1, m), 1)
    d0 = (xs - x0) ** 2 + (ys - y0) ** 2 + (zs - z0) ** 2
    cx_ref[...] = jnp.broadcast_to(x0, cx_ref.shape)
    cy_ref[...] = jnp.broadcast_to(y0, cy_ref.shape)
    cz_ref[...] = jnp.broadcast_to(z0, cz_ref.shape)

    def body(i, dists):
        mx = jnp.max(dists, axis=1, keepdims=True)
        sel = jnp.min(jnp.where(dists == mx, iota, n), axis=1, keepdims=True)
        mask = (iota == sel).astype(_F32)
        xc = jnp.sum(xs * mask, axis=1, keepdims=True)
        yc = jnp.sum(ys * mask, axis=1, keepdims=True)
        zc = jnp.sum(zs * mask, axis=1, keepdims=True)
        dnew = (xs - xc) ** 2 + (ys - yc) ** 2 + (zs - zc) ** 2
        hit = lane_m == i
        cx_ref[...] = jnp.where(hit, xc, cx_ref[...])
        cy_ref[...] = jnp.where(hit, yc, cy_ref[...])
        cz_ref[...] = jnp.where(hit, zc, cz_ref[...])
        return jnp.minimum(dists, dnew)

    jax.lax.fori_loop(1, m, body, d0)


def _knn_idx_kernel(cen_ref, xyzT_ref, i_out_ref, d_ref, *, n, m, k):
    cen = cen_ref[...]
    sxyzT = xyzT_ref[...]
    D0 = ((cen[:, 0:1] - sxyzT[0:1, :]) ** 2
          + (cen[:, 1:2] - sxyzT[1:2, :]) ** 2)
    d_ref[...] = D0 + (cen[:, 2:3] - sxyzT[2:3, :]) ** 2
    lane_mn = jax.lax.broadcasted_iota(jnp.int32, (m, n), 1)

    def body(j, _):
        D = d_ref[...]
        oh, _, idx = _first_min_onehot(D, lane_mn, n)
        d_ref[...] = D + oh * _BIG
        i_out_ref[j] = idx
        return 0

    jax.lax.fori_loop(0, k, body, 0)


def _sc_gather(table, idx, D):
    info = plsc.get_sparse_core_info()
    NW = info.num_cores * info.num_subcores
    B = idx.shape[0]
    b_per_w = B // NW
    mesh = plsc.VectorSubcoreMesh(core_axis_name="c", subcore_axis_name="s")

    ch = min(b_per_w, 512)
    n_ch = b_per_w // ch

    @functools.partial(
        pl.kernel, mesh=mesh,
        out_type=jax.ShapeDtypeStruct((B, D), jnp.float32),
        scratch_types=[
            pltpu.VMEM((ch,), jnp.int32),
            pltpu.VMEM((ch, D), jnp.float32),
            pltpu.SemaphoreType.DMA,
        ],
    )
    def gk(table_hbm, idx_hbm, out_hbm, idx_v, rows_v, sem):
        wid = lax.axis_index("s") * info.num_cores + lax.axis_index("c")
        base = wid * b_per_w
        for c in range(n_ch):
            off = base + c * ch
            pltpu.sync_copy(idx_hbm.at[pl.ds(off, ch)], idx_v)
            pltpu.async_copy(table_hbm.at[idx_v], rows_v, sem).wait()
            pltpu.sync_copy(rows_v, out_hbm.at[pl.ds(off, ch)])

    return gk(table, idx)


def _sa_mlp_kernel(rows_ref, cen_ref,
                   w1_ref, b1_ref, w2_ref, b2_ref, w3_ref, b3_ref,
                   fout_ref, *, m, k, cf, cpad):
    cen = cen_ref[...]
    cenpad = jnp.concatenate(
        [jnp.zeros((m, cf), _F32), cen,
         jnp.zeros((m, cpad - cf - 3), _F32)], axis=1)
    h = rows_ref[...].reshape(k, m, cpad) - cenpad[None, :, :]
    a = jnp.maximum(_dot(h.reshape(k * m, cpad), w1_ref[...]) + b1_ref[...], 0.0)
    a = jnp.maximum(_dot(a, w2_ref[...]) + b2_ref[...], 0.0)
    a = jnp.maximum(_dot(a, w3_ref[...]) + b3_ref[...], 0.0)
    fout_ref[...] = jnp.max(a.reshape(k, m, -1), axis=0)


def _sa_kernel(featxyz_ref, cen_ref, xyzT_ref,
               w1_ref, b1_ref, w2_ref, b2_ref, w3_ref, b3_ref,
               fout_ref, d_ref,
               *, n, m, k, cf):
    cen = cen_ref[...]
    sxyzT = xyzT_ref[...]
    D0 = ((cen[:, 0:1] - sxyzT[0:1, :]) ** 2
          + (cen[:, 1:2] - sxyzT[1:2, :]) ** 2)
    d_ref[...] = D0 + (cen[:, 2:3] - sxyzT[2:3, :]) ** 2

    lane_mn = jax.lax.broadcasted_iota(jnp.int32, (m, n), 1)
    cenpad = jnp.concatenate([jnp.zeros((m, cf), _F32), cen], axis=1)
    featxyz = featxyz_ref[...]
    w1, b1 = w1_ref[...], b1_ref[...]
    w2, b2 = w2_ref[...], b2_ref[...]
    w3, b3 = w3_ref[...], b3_ref[...]

    def nbr_body(_, acc):
        D = d_ref[...]
        oh, _, idx = _first_min_onehot(D, lane_mn, n)
        d_ref[...] = D + oh * _BIG
        g = _dot(oh, featxyz) - cenpad
        a = jnp.maximum(_dot(g, w1) + b1, 0.0)
        a = jnp.maximum(_dot(a, w2) + b2, 0.0)
        a = jnp.maximum(_dot(a, w3) + b3, 0.0)
        return jnp.maximum(acc, a)

    cout = w3.shape[1]
    fout_ref[...] = jax.lax.fori_loop(
        0, k, nbr_body, jnp.zeros((m, cout), _F32))


def _sa(layers, feat, xyz, m, k):
    G, n, cf = feat.shape
    (w1, b1), (w2, b2), (w3, b3) = layers
    cout = w3.shape[1]
    featxyz = jnp.concatenate([feat, xyz], axis=2)
    xyzT = jnp.transpose(xyz, (0, 2, 1))
    cx, cy, cz = pl.pallas_call(
        functools.partial(_fps_kernel, n=n, m=m),
        out_shape=(jax.ShapeDtypeStruct((G, m), _F32),) * 3,
    )(xyzT[:, 0], xyzT[:, 1], xyzT[:, 2])
    cen = jnp.stack([cx, cy, cz], axis=2)

    if n >= 2048:
        idx = jax.vmap(pl.pallas_call(
            functools.partial(_knn_idx_kernel, n=n, m=m, k=k),
            out_shape=jax.ShapeDtypeStruct((k, m, 1), jnp.int32),
            scratch_shapes=[pltpu.VMEM((m, n), _F32)],
        ), in_axes=(0, 0))(cen, xyzT)
        cpad = 128 * ((cf + 3 + 127) // 128)
        table = jnp.pad(featxyz, ((0, 0), (0, 0), (0, cpad - cf - 3)))
        idx_flat = (idx.reshape(G, k * m)
                    + (jnp.arange(G, dtype=jnp.int32) * n)[:, None]).reshape(-1)
        rows = _sc_gather(table.reshape(G * n, cpad), idx_flat, cpad)
        fn = pl.pallas_call(
            functools.partial(_sa_mlp_kernel, m=m, k=k, cf=cf, cpad=cpad),
            out_shape=jax.ShapeDtypeStruct((m, cout), _F32),
        )
        w1p = jnp.pad(w1, ((0, cpad - cf - 3), (0, 0)))
        fout = jax.vmap(fn, in_axes=(0, 0) + (None,) * 6)(
            rows.reshape(G, k * m, cpad), cen,
            w1p, b1.reshape(1, -1), w2, b2.reshape(1, -1),
            w3, b3.reshape(1, -1))
        return fout, cen

    fn = pl.pallas_call(
        functools.partial(_sa_kernel, n=n, m=m, k=k, cf=cf),
        out_shape=jax.ShapeDtypeStruct((m, cout), _F32),
        scratch_shapes=[pltpu.VMEM((m, n), _F32)],
    )
    fout = jax.vmap(fn, in_axes=(0, 0, 0) + (None,) * 6)(
        featxyz, cen, xyzT,
        w1, b1.reshape(1, -1), w2, b2.reshape(1, -1), w3, b3.reshape(1, -1))
    return fout, cen


def _lpt_kernel(fcur_ref, fsrc_ref, qxyz_ref, sxyzT_ref,
                wq_ref, wk_ref, wv_ref, out_ref, d_ref, s_ref, i_ref,
                *, m, n, k, c):
    qxyz = qxyz_ref[...]
    sxyzT = sxyzT_ref[...]
    D0 = ((qxyz[:, 0:1] - sxyzT[0:1, :]) ** 2
          + (qxyz[:, 1:2] - sxyzT[1:2, :]) ** 2)
    d_ref[...] = D0 + (qxyz[:, 2:3] - sxyzT[2:3, :]) ** 2
    lane_mn = jax.lax.broadcasted_iota(jnp.int32, (m, n), 1)

    q = _dot(fcur_ref[...], wq_ref[...])
    K = _dot(fsrc_ref[...], wk_ref[...])
    V = _dot(fsrc_ref[...], wv_ref[...])
    S = jax.lax.dot_general(q, K, (((1,), (1,)), ((), ())),
                            preferred_element_type=_F32,
                            precision=_PREC)

    def sel_body(j, _):
        D = d_ref[...]
        oh, _, idx = _first_min_onehot(D, lane_mn, n)
        d_ref[...] = D + oh * _BIG
        s_ref[j] = jnp.sum(S * oh, axis=1, keepdims=True)
        i_ref[j] = idx
        return 0

    jax.lax.fori_loop(0, k, sel_body, 0)

    s = s_ref[...] / np.sqrt(c)
    e = jnp.exp(s - jnp.max(s, axis=0, keepdims=True))
    s_ref[...] = e / jnp.sum(e, axis=0, keepdims=True)

    def mix_body(j, A):
        return A + s_ref[j] * (lane_mn == i_ref[j]).astype(_F32)

    A = jax.lax.fori_loop(0, k, mix_body, jnp.zeros((m, n), _F32))
    out_ref[...] = _dot(A, V)


def _lpt(p, f_cur, f_src, xyz_cur, xyz_src, k):
    B, m, c = f_cur.shape
    n = f_src.shape[1]
    sxyzT = jnp.transpose(xyz_src, (0, 2, 1))
    fn = pl.pallas_call(
        functools.partial(_lpt_kernel, m=m, n=n, k=k, c=c),
        out_shape=jax.ShapeDtypeStruct((m, c), _F32),
        scratch_shapes=[pltpu.VMEM((m, n), _F32),
                        pltpu.VMEM((k, m, 1), _F32),
                        pltpu.VMEM((k, m, 1), jnp.int32)],
    )
    return jax.vmap(fn, in_axes=(0, 0, 0, 0, None, None, None))(
        f_cur, f_src, xyz_cur, sxyzT, p['Wq'], p['Wk'], p['Wv'])


def _lstm3_kernel(*refs):
    for lvl in range(3):
        fb_ref, ff_ref, h_ref, c_ref = refs[4 * lvl:4 * lvl + 4]
        wx_ref, wh_ref, b_ref = refs[12 + 3 * lvl:15 + 3 * lvl]
        hout_ref, cout_ref = refs[21 + 2 * lvl:23 + 2 * lvl]
        hdim = h_ref.shape[1]
        x = jnp.concatenate([fb_ref[...], ff_ref[...]], axis=1)
        g = _dot(x, wx_ref[...]) + _dot(h_ref[...], wh_ref[...]) + b_ref[...]
        i = jax.nn.sigmoid(g[:, 0:hdim])
        f = jax.nn.sigmoid(g[:, hdim:2 * hdim])
        gg = jnp.tanh(g[:, 2 * hdim:3 * hdim])
        o = jax.nn.sigmoid(g[:, 3 * hdim:4 * hdim])
        cn = f * c_ref[...] + i * gg
        hout_ref[...] = o * jnp.tanh(cn)
        cout_ref[...] = cn


def _lstm3(p, st, fbff):
    H1, C1, H2, C2, H3, C3 = st
    fb1, ff1, fb2, ff2, fb3, ff3 = fbff
    B = H1.shape[0]
    shapes = tuple(jax.ShapeDtypeStruct(h.shape[1:], _F32)
                   for h in (H1, H1, H2, H2, H3, H3))
    fn = pl.pallas_call(_lstm3_kernel, out_shape=shapes)
    ws = []
    for name in ('lstm1', 'lstm2', 'lstm3'):
        ws += [p[name]['Wx'], p[name]['Wh'], p[name]['b'].reshape(1, -1)]
    return jax.vmap(fn, in_axes=(0,) * 12 + (None,) * 9)(
        fb1, ff1, H1, C1, fb2, ff2, H2, C2, fb3, ff3, H3, C3, *ws)


def _interp(xc, posf, poscT, d_ref, m, n, k):
    D0 = ((posf[:, 0:1] - poscT[0:1, :]) ** 2
          + (posf[:, 1:2] - poscT[1:2, :]) ** 2)
    d_ref[...] = D0 + (posf[:, 2:3] - poscT[2:3, :]) ** 2
    lane_mn = jax.lax.broadcasted_iota(jnp.int32, (m, n), 1)

    def body(_, carry):
        W, wsum = carry
        D = d_ref[...]
        oh, mn, idx = _first_min_onehot(D, lane_mn, n)
        d_ref[...] = D + oh * _BIG
        w = 1.0 / (mn + 1e-2)
        return W + w * oh, wsum + w

    W, wsum = jax.lax.fori_loop(
        0, k, body, (jnp.zeros((m, n), _F32), jnp.zeros((m, 1), _F32)))
    return _dot(W, xc) / wsum


def _fp_kernel(xc_ref, posf_ref, poscT_ref, xskip_ref,
               w1_ref, b1_ref, w2_ref, b2_ref, out_ref, d_ref, *, m, n, k):
    interp = _interp(xc_ref[...], posf_ref[...], poscT_ref[...], d_ref, m, n, k)
    h = jnp.concatenate([interp, xskip_ref[...]], axis=1)
    h = jnp.maximum(_dot(h, w1_ref[...]) + b1_ref[...], 0.0)
    h = jnp.maximum(_dot(h, w2_ref[...]) + b2_ref[...], 0.0)
    out_ref[...] = h


def _fp(layers, x_c, pos_c, x_skip, pos_f, k):
    B, m, _ = pos_f.shape
    n = pos_c.shape[1]
    (w1, b1), (w2, b2) = layers
    poscT = jnp.transpose(pos_c, (0, 2, 1))
    fn = pl.pallas_call(
        functools.partial(_fp_kernel, m=m, n=n, k=k),
        out_shape=jax.ShapeDtypeStruct((m, w2.shape[1]), _F32),
        scratch_shapes=[pltpu.VMEM((m, n), _F32)],
    )
    return jax.vmap(fn, in_axes=(0, 0, 0, 0, None, None, None, None))(
        x_c, pos_f, poscT, x_skip, w1, b1.reshape(1, -1), w2, b2.reshape(1, -1))


def _fpns_cls_kernel(xc_ref, posf_ref, poscT_ref,
                     w1_ref, b1_ref, w2_ref, b2_ref,
                     c1_ref, c2_ref, c3_ref, c4_ref, out_ref, d_ref,
                     *, m, n, k):
    interp = _interp(xc_ref[...], posf_ref[...], poscT_ref[...], d_ref, m, n, k)
    h = jnp.maximum(_dot(interp, w1_ref[...]) + b1_ref[...], 0.0)
    h = jnp.maximum(_dot(h, w2_ref[...]) + b2_ref[...], 0.0)
    h = _dot(h, c1_ref[...])
    h = _dot(h, c2_ref[...])
    h = _dot(h, c3_ref[...])
    h = _dot(h, c4_ref[...])
    out_ref[...] = posf_ref[...] + h


def _fpns_cls(layers, cls, x_c, pos_c, pos_f, k):
    B, m, _ = pos_f.shape
    n = pos_c.shape[1]
    (w1, b1), (w2, b2) = layers
    c1, c2, c3, c4 = cls
    poscT = jnp.transpose(pos_c, (0, 2, 1))
    fn = pl.pallas_call(
        functools.partial(_fpns_cls_kernel, m=m, n=n, k=k),
        out_shape=jax.ShapeDtypeStruct((m, 3), _F32),
        scratch_shapes=[pltpu.VMEM((m, n), _F32)],
    )
    return jax.vmap(fn, in_axes=(0, 0, 0) + (None,) * 8)(
        x_c, pos_f, poscT, w1, b1.reshape(1, -1), w2, b2.reshape(1, -1),
        c1, c2, c3, c4)


def kernel(input_xyz, num_pred, params):
    p = params
    T, B, _, N = input_xyz.shape
    frames = jnp.transpose(input_xyz, (0, 1, 3, 2))
    N1, N2, N3 = N // 16, N // 32, N // 64

    def encode(fr):
        f1, x1 = _sa(p['sa1'], fr, fr, N1, 32)
        f2, x2 = _sa(p['sa2'], f1, x1, N2, 16)
        f3, x3 = _sa(p['sa3'], f2, x2, N3, 8)
        return (f1, x1, f2, x2, f3, x3)

    e_all = encode(frames.reshape(T * B, N, 3))
    encs = [tuple(a.reshape((T, B) + a.shape[1:])[t] for a in e_all)
            for t in range(T)]

    st = (jnp.zeros((B, N1, 128), _F32), jnp.zeros((B, N1, 128), _F32),
          jnp.zeros((B, N2, 256), _F32), jnp.zeros((B, N2, 256), _F32),
          jnp.zeros((B, N3, 512), _F32), jnp.zeros((B, N3, 512), _F32))

    def lpt_all(pairs, p_l, k_l, fi, pi):
        f_cur = jnp.concatenate([c[fi] for c, _ in pairs], axis=0)
        f_src = jnp.concatenate([s[fi] for _, s in pairs], axis=0)
        x_cur = jnp.concatenate([c[pi] for c, _ in pairs], axis=0)
        x_src = jnp.concatenate([s[pi] for _, s in pairs], axis=0)
        out = _lpt(p_l, f_cur, f_src, x_cur, x_src, k_l)
        return out.reshape((len(pairs), B) + out.shape[1:])

    def lpt_levels(pairs):
        a1 = lpt_all(pairs, p['gat1'], 16, 0, 1)
        a2 = lpt_all(pairs, p['gat2'], 16, 2, 3)
        a3 = lpt_all(pairs, p['gat3'], 8, 4, 5)
        return a1, a2, a3

    pairs = []
    for t in range(T):
        prev = encs[t - 1] if t > 0 else encs[0]
        nxt = encs[t + 1] if t < T - 1 else encs[t]
        pairs += [(encs[t], prev), (encs[t], nxt)]
    a1, a2, a3 = lpt_levels(pairs)
    for t in range(T):
        st = _lstm3(p, st, (a1[2 * t], a1[2 * t + 1], a2[2 * t],
                            a2[2 * t + 1], a3[2 * t], a3[2 * t + 1]))

    def decode(st, e, fine_xyz):
        H1, _, H2, _, H3, _ = st
        x2 = _fp(p['fp32'], H3, e[5], H2, e[3], 8)
        x1 = _fp(p['fp21'], x2, e[3], H1, e[1], 16)
        return _fpns_cls(p['fp10'], p['cls'], x1, e[1], fine_xyz, 32)

    num_steps = 2
    pc_next = decode(st, encs[-1], frames[-1])
    preds = [pc_next]
    for _ in range(1, num_steps):
        e_new = encode(pc_next)
        b1, b2, b3 = lpt_levels([(e_new, encs[-1]), (e_new, e_new)])
        st = _lstm3(p, st, (b1[0], b1[1], b2[0], b2[1], b3[0], b3[1]))
        encs.append(e_new)
        pc_next = decode(st, e_new, pc_next)
        preds.append(pc_next)
    return jnp.stack(preds)


# speedup vs baseline: 1.1972x; 1.0343x over previous
import functools

import jax
import jax.numpy as jnp
import numpy as np
from jax import lax
from jax.experimental import pallas as pl
from jax.experimental.pallas import tpu as pltpu
from jax.experimental.pallas import tpu_sc as plsc

_F32 = jnp.float32
_BIG = 3.0e38
_PREC = jax.lax.Precision.DEFAULT


def _dot(a, b):
    return jax.lax.dot_general(a, b, (((1,), (0,)), ((), ())),
                               preferred_element_type=_F32, precision=_PREC)


def _first_min_onehot(D, lane_iota, n):
    mn = jnp.min(D, axis=1, keepdims=True)
    idx = jnp.min(jnp.where(D == mn, lane_iota, n), axis=1, keepdims=True)
    oh = (lane_iota == idx).astype(_F32)
    return oh, mn, idx


def _fps_kernel(xs_ref, ys_ref, zs_ref, cx_ref, cy_ref, cz_ref, *, n, m):
    xs, ys, zs = xs_ref[...], ys_ref[...], zs_ref[...]
    x0, y0, z0 = xs[:, 0:1], ys[:, 0:1], zs[:, 0:1]
    iota = jax.lax.broadcasted_iota(jnp.int32, (1, n), 1)
    lane_m = jax.lax.broadcasted_iota(jnp.int32, (1, m), 1)
    d0 = (xs - x0) ** 2 + (ys - y0) ** 2 + (zs - z0) ** 2
    cx_ref[...] = jnp.broadcast_to(x0, cx_ref.shape)
    cy_ref[...] = jnp.broadcast_to(y0, cy_ref.shape)
    cz_ref[...] = jnp.broadcast_to(z0, cz_ref.shape)

    def body(i, dists):
        mx = jnp.max(dists, axis=1, keepdims=True)
        sel = jnp.min(jnp.where(dists == mx, iota, n), axis=1, keepdims=True)
        mask = (iota == sel).astype(_F32)
        xc = jnp.sum(xs * mask, axis=1, keepdims=True)
        yc = jnp.sum(ys * mask, axis=1, keepdims=True)
        zc = jnp.sum(zs * mask, axis=1, keepdims=True)
        dnew = (xs - xc) ** 2 + (ys - yc) ** 2 + (zs - zc) ** 2
        hit = lane_m == i
        cx_ref[...] = jnp.where(hit, xc, cx_ref[...])
        cy_ref[...] = jnp.where(hit, yc, cy_ref[...])
        cz_ref[...] = jnp.where(hit, zc, cz_ref[...])
        return jnp.minimum(dists, dnew)

    jax.lax.fori_loop(1, m, body, d0)


def _knn_idx_kernel(cen_ref, xyzT_ref, i_out_ref, d_ref, *, n, m, k):
    cen = cen_ref[...]
    sxyzT = xyzT_ref[...]
    D0 = ((cen[:, 0:1] - sxyzT[0:1, :]) ** 2
          + (cen[:, 1:2] - sxyzT[1:2, :]) ** 2)
    d_ref[...] = D0 + (cen[:, 2:3] - sxyzT[2:3, :]) ** 2
    lane_mn = jax.lax.broadcasted_iota(jnp.int32, (m, n), 1)

    def body(j, _):
        D = d_ref[...]
        oh, _, idx = _first_min_onehot(D, lane_mn, n)
        d_ref[...] = D + oh * _BIG
        i_out_ref[j] = idx
        return 0

    jax.lax.fori_loop(0, k, body, 0)


def _sc_gather(table, idx, D):
    info = plsc.get_sparse_core_info()
    NW = info.num_cores * info.num_subcores
    B = idx.shape[0]
    b_per_w = B // NW
    mesh = plsc.VectorSubcoreMesh(core_axis_name="c", subcore_axis_name="s")

    ch = min(b_per_w, 512)
    n_ch = b_per_w // ch

    @functools.partial(
        pl.kernel, mesh=mesh,
        out_type=jax.ShapeDtypeStruct((B, D), jnp.float32),
        scratch_types=[
            pltpu.VMEM((ch,), jnp.int32),
            pltpu.VMEM((ch, D), jnp.float32),
            pltpu.SemaphoreType.DMA,
        ],
    )
    def gk(table_hbm, idx_hbm, out_hbm, idx_v, rows_v, sem):
        wid = lax.axis_index("s") * info.num_cores + lax.axis_index("c")
        base = wid * b_per_w
        for c in range(n_ch):
            off = base + c * ch
            pltpu.sync_copy(idx_hbm.at[pl.ds(off, ch)], idx_v)
            pltpu.async_copy(table_hbm.at[idx_v], rows_v, sem).wait()
            pltpu.sync_copy(rows_v, out_hbm.at[pl.ds(off, ch)])

    return gk(table, idx)


def _sa_mlp_kernel(rows_ref, cen_ref,
                   w1_ref, b1_ref, w2_ref, b2_ref, w3_ref, b3_ref,
                   fout_ref, *, m, k, cf, cpad):
    cen = cen_ref[...]
    cenpad = jnp.concatenate(
        [jnp.zeros((m, cf), _F32), cen,
         jnp.zeros((m, cpad - cf - 3), _F32)], axis=1)
    h = rows_ref[...].reshape(k, m, cpad) - cenpad[None, :, :]
    a = jnp.maximum(_dot(h.reshape(k * m, cpad), w1_ref[...]) + b1_ref[...], 0.0)
    a = jnp.maximum(_dot(a, w2_ref[...]) + b2_ref[...], 0.0)
    a = jnp.maximum(_dot(a, w3_ref[...]) + b3_ref[...], 0.0)
    fout_ref[...] = jnp.max(a.reshape(k, m, -1), axis=0)


def _sa_kernel(featxyz_ref, cen_ref, xyzT_ref,
               w1_ref, b1_ref, w2_ref, b2_ref, w3_ref, b3_ref,
               fout_ref, d_ref,
               *, n, m, k, cf):
    cen = cen_ref[...]
    sxyzT = xyzT_ref[...]
    D0 = ((cen[:, 0:1] - sxyzT[0:1, :]) ** 2
          + (cen[:, 1:2] - sxyzT[1:2, :]) ** 2)
    d_ref[...] = D0 + (cen[:, 2:3] - sxyzT[2:3, :]) ** 2

    lane_mn = jax.lax.broadcasted_iota(jnp.int32, (m, n), 1)
    cenpad = jnp.concatenate([jnp.zeros((m, cf), _F32), cen], axis=1)
    featxyz = featxyz_ref[...]
    w1, b1 = w1_ref[...], b1_ref[...]
    w2, b2 = w2_ref[...], b2_ref[...]
    w3, b3 = w3_ref[...], b3_ref[...]

    def nbr_body(_, acc):
        D = d_ref[...]
        oh, _, idx = _first_min_onehot(D, lane_mn, n)
        d_ref[...] = D + oh * _BIG
        g = _dot(oh, featxyz) - cenpad
        a = jnp.maximum(_dot(g, w1) + b1, 0.0)
        a = jnp.maximum(_dot(a, w2) + b2, 0.0)
        a = jnp.maximum(_dot(a, w3) + b3, 0.0)
        return jnp.maximum(acc, a)

    cout = w3.shape[1]
    fout_ref[...] = jax.lax.fori_loop(
        0, k, nbr_body, jnp.zeros((m, cout), _F32))


def _sa(layers, feat, xyz, m, k):
    G, n, cf = feat.shape
    (w1, b1), (w2, b2), (w3, b3) = layers
    cout = w3.shape[1]
    featxyz = jnp.concatenate([feat, xyz], axis=2)
    xyzT = jnp.transpose(xyz, (0, 2, 1))
    cx, cy, cz = pl.pallas_call(
        functools.partial(_fps_kernel, n=n, m=m),
        out_shape=(jax.ShapeDtypeStruct((G, m), _F32),) * 3,
    )(xyzT[:, 0], xyzT[:, 1], xyzT[:, 2])
    cen = jnp.stack([cx, cy, cz], axis=2)

    if n >= 2048:
        idx = jax.vmap(pl.pallas_call(
            functools.partial(_knn_idx_kernel, n=n, m=m, k=k),
            out_shape=jax.ShapeDtypeStruct((k, m, 1), jnp.int32),
            scratch_shapes=[pltpu.VMEM((m, n), _F32)],
        ), in_axes=(0, 0))(cen, xyzT)
        cpad = 128 * ((cf + 3 + 127) // 128)
        table = jnp.pad(featxyz, ((0, 0), (0, 0), (0, cpad - cf - 3)))
        idx_flat = (idx.reshape(G, k * m)
                    + (jnp.arange(G, dtype=jnp.int32) * n)[:, None]).reshape(-1)
        rows = _sc_gather(table.reshape(G * n, cpad), idx_flat, cpad)
        fn = pl.pallas_call(
            functools.partial(_sa_mlp_kernel, m=m, k=k, cf=cf, cpad=cpad),
            out_shape=jax.ShapeDtypeStruct((m, cout), _F32),
        )
        w1p = jnp.pad(w1, ((0, cpad - cf - 3), (0, 0)))
        fout = jax.vmap(fn, in_axes=(0, 0) + (None,) * 6)(
            rows.reshape(G, k * m, cpad), cen,
            w1p, b1.reshape(1, -1), w2, b2.reshape(1, -1),
            w3, b3.reshape(1, -1))
        return fout, cen

    fn = pl.pallas_call(
        functools.partial(_sa_kernel, n=n, m=m, k=k, cf=cf),
        out_shape=jax.ShapeDtypeStruct((m, cout), _F32),
        scratch_shapes=[pltpu.VMEM((m, n), _F32)],
    )
    fout = jax.vmap(fn, in_axes=(0, 0, 0) + (None,) * 6)(
        featxyz, cen, xyzT,
        w1, b1.reshape(1, -1), w2, b2.reshape(1, -1), w3, b3.reshape(1, -1))
    return fout, cen


def _lpt_kernel(fcur_ref, fsrc_ref, qxyz_ref, sxyzT_ref,
                wq_ref, wk_ref, wv_ref, out_ref, d_ref, s_ref, i_ref,
                *, m, n, k, c):
    qxyz = qxyz_ref[...]
    sxyzT = sxyzT_ref[...]
    D0 = ((qxyz[:, 0:1] - sxyzT[0:1, :]) ** 2
          + (qxyz[:, 1:2] - sxyzT[1:2, :]) ** 2)
    d_ref[...] = D0 + (qxyz[:, 2:3] - sxyzT[2:3, :]) ** 2
    lane_mn = jax.lax.broadcasted_iota(jnp.int32, (m, n), 1)

    q = _dot(fcur_ref[...], wq_ref[...])
    K = _dot(fsrc_ref[...], wk_ref[...])
    V = _dot(fsrc_ref[...], wv_ref[...])
    S = jax.lax.dot_general(q, K, (((1,), (1,)), ((), ())),
                            preferred_element_type=_F32,
                            precision=_PREC)

    def sel_body(j, _):
        D = d_ref[...]
        oh, _, idx = _first_min_onehot(D, lane_mn, n)
        d_ref[...] = D + oh * _BIG
        s_ref[j] = jnp.sum(S * oh, axis=1, keepdims=True)
        i_ref[j] = idx
        return 0

    jax.lax.fori_loop(0, k, sel_body, 0)

    s = s_ref[...] / np.sqrt(c)
    e = jnp.exp(s - jnp.max(s, axis=0, keepdims=True))
    s_ref[...] = e / jnp.sum(e, axis=0, keepdims=True)

    def mix_body(j, A):
        return A + s_ref[j] * (lane_mn == i_ref[j]).astype(_F32)

    A = jax.lax.fori_loop(0, k, mix_body, jnp.zeros((m, n), _F32))
    out_ref[...] = _dot(A, V)


def _lpt(p, f_cur, f_src, xyz_cur, xyz_src, k):
    B, m, c = f_cur.shape
    n = f_src.shape[1]
    sxyzT = jnp.transpose(xyz_src, (0, 2, 1))
    fn = pl.pallas_call(
        functools.partial(_lpt_kernel, m=m, n=n, k=k, c=c),
        out_shape=jax.ShapeDtypeStruct((m, c), _F32),
        scratch_shapes=[pltpu.VMEM((m, n), _F32),
                        pltpu.VMEM((k, m, 1), _F32),
                        pltpu.VMEM((k, m, 1), jnp.int32)],
    )
    return jax.vmap(fn, in_axes=(0, 0, 0, 0, None, None, None))(
        f_cur, f_src, xyz_cur, sxyzT, p['Wq'], p['Wk'], p['Wv'])


def _lstm3_kernel(*refs):
    for lvl in range(3):
        fb_ref, ff_ref, h_ref, c_ref = refs[4 * lvl:4 * lvl + 4]
        wx_ref, wh_ref, b_ref = refs[12 + 3 * lvl:15 + 3 * lvl]
        hout_ref, cout_ref = refs[21 + 2 * lvl:23 + 2 * lvl]
        hdim = h_ref.shape[1]
        x = jnp.concatenate([fb_ref[...], ff_ref[...]], axis=1)
        g = _dot(x, wx_ref[...]) + _dot(h_ref[...], wh_ref[...]) + b_ref[...]
        i = jax.nn.sigmoid(g[:, 0:hdim])
        f = jax.nn.sigmoid(g[:, hdim:2 * hdim])
        gg = jnp.tanh(g[:, 2 * hdim:3 * hdim])
        o = jax.nn.sigmoid(g[:, 3 * hdim:4 * hdim])
        cn = f * c_ref[...] + i * gg
        hout_ref[...] = o * jnp.tanh(cn)
        cout_ref[...] = cn


def _lstm3(p, st, fbff):
    H1, C1, H2, C2, H3, C3 = st
    fb1, ff1, fb2, ff2, fb3, ff3 = fbff
    B = H1.shape[0]
    shapes = tuple(jax.ShapeDtypeStruct(h.shape[1:], _F32)
                   for h in (H1, H1, H2, H2, H3, H3))
    fn = pl.pallas_call(_lstm3_kernel, out_shape=shapes)
    ws = []
    for name in ('lstm1', 'lstm2', 'lstm3'):
        ws += [p[name]['Wx'], p[name]['Wh'], p[name]['b'].reshape(1, -1)]
    return jax.vmap(fn, in_axes=(0,) * 12 + (None,) * 9)(
        fb1, ff1, H1, C1, fb2, ff2, H2, C2, fb3, ff3, H3, C3, *ws)


def _interp(xc, posf, poscT, d_ref, w_ref, m, n, k):
    D0 = ((posf[:, 0:1] - poscT[0:1, :]) ** 2
          + (posf[:, 1:2] - poscT[1:2, :]) ** 2)
    d_ref[...] = D0 + (posf[:, 2:3] - poscT[2:3, :]) ** 2
    w_ref[...] = jnp.zeros((m, n), _F32)
    lane_mn = jax.lax.broadcasted_iota(jnp.int32, (m, n), 1)

    def body(_, wsum):
        D = d_ref[...]
        oh, mn, idx = _first_min_onehot(D, lane_mn, n)
        d_ref[...] = D + oh * _BIG
        w = 1.0 / (mn + 1e-2)
        w_ref[...] = w_ref[...] + w * oh
        return wsum + w

    wsum = jax.lax.fori_loop(0, k, body, jnp.zeros((m, 1), _F32))
    return _dot(w_ref[...], xc) / wsum


def _fp_kernel(xc_ref, posf_ref, poscT_ref, xskip_ref,
               w1_ref, b1_ref, w2_ref, b2_ref, out_ref, d_ref, w_ref,
               *, m, n, k):
    interp = _interp(xc_ref[...], posf_ref[...], poscT_ref[...], d_ref, w_ref,
                     m, n, k)
    h = jnp.concatenate([interp, xskip_ref[...]], axis=1)
    h = jnp.maximum(_dot(h, w1_ref[...]) + b1_ref[...], 0.0)
    h = jnp.maximum(_dot(h, w2_ref[...]) + b2_ref[...], 0.0)
    out_ref[...] = h


def _fp(layers, x_c, pos_c, x_skip, pos_f, k):
    B, m, _ = pos_f.shape
    n = pos_c.shape[1]
    (w1, b1), (w2, b2) = layers
    poscT = jnp.transpose(pos_c, (0, 2, 1))
    fn = pl.pallas_call(
        functools.partial(_fp_kernel, m=m, n=n, k=k),
        out_shape=jax.ShapeDtypeStruct((m, w2.shape[1]), _F32),
        scratch_shapes=[pltpu.VMEM((m, n), _F32), pltpu.VMEM((m, n), _F32)],
    )
    return jax.vmap(fn, in_axes=(0, 0, 0, 0, None, None, None, None))(
        x_c, pos_f, poscT, x_skip, w1, b1.reshape(1, -1), w2, b2.reshape(1, -1))


def _fpns_cls_kernel(xc_ref, posf_ref, poscT_ref,
                     w1_ref, b1_ref, w2_ref, b2_ref,
                     c1_ref, c2_ref, c3_ref, c4_ref, out_ref, d_ref, w_ref,
                     *, m, n, k):
    interp = _interp(xc_ref[...], posf_ref[...], poscT_ref[...], d_ref, w_ref,
                     m, n, k)
    h = jnp.maximum(_dot(interp, w1_ref[...]) + b1_ref[...], 0.0)
    h = jnp.maximum(_dot(h, w2_ref[...]) + b2_ref[...], 0.0)
    h = _dot(h, c1_ref[...])
    h = _dot(h, c2_ref[...])
    h = _dot(h, c3_ref[...])
    h = _dot(h, c4_ref[...])
    out_ref[...] = posf_ref[...] + h


def _fpns_cls(layers, cls, x_c, pos_c, pos_f, k):
    B, m, _ = pos_f.shape
    n = pos_c.shape[1]
    (w1, b1), (w2, b2) = layers
    c1, c2, c3, c4 = cls
    poscT = jnp.transpose(pos_c, (0, 2, 1))
    fn = pl.pallas_call(
        functools.partial(_fpns_cls_kernel, m=m, n=n, k=k),
        out_shape=jax.ShapeDtypeStruct((m, 3), _F32),
        scratch_shapes=[pltpu.VMEM((m, n), _F32), pltpu.VMEM((m, n), _F32)],
    )
    return jax.vmap(fn, in_axes=(0, 0, 0) + (None,) * 8)(
        x_c, pos_f, poscT, w1, b1.reshape(1, -1), w2, b2.reshape(1, -1),
        c1, c2, c3, c4)


def kernel(input_xyz, num_pred, params):
    p = params
    T, B, _, N = input_xyz.shape
    frames = jnp.transpose(input_xyz, (0, 1, 3, 2))
    N1, N2, N3 = N // 16, N // 32, N // 64

    def encode(fr):
        f1, x1 = _sa(p['sa1'], fr, fr, N1, 32)
        f2, x2 = _sa(p['sa2'], f1, x1, N2, 16)
        f3, x3 = _sa(p['sa3'], f2, x2, N3, 8)
        return (f1, x1, f2, x2, f3, x3)

    e_all = encode(frames.reshape(T * B, N, 3))
    encs = [tuple(a.reshape((T, B) + a.shape[1:])[t] for a in e_all)
            for t in range(T)]

    st = (jnp.zeros((B, N1, 128), _F32), jnp.zeros((B, N1, 128), _F32),
          jnp.zeros((B, N2, 256), _F32), jnp.zeros((B, N2, 256), _F32),
          jnp.zeros((B, N3, 512), _F32), jnp.zeros((B, N3, 512), _F32))

    def lpt_all(pairs, p_l, k_l, fi, pi):
        f_cur = jnp.concatenate([c[fi] for c, _ in pairs], axis=0)
        f_src = jnp.concatenate([s[fi] for _, s in pairs], axis=0)
        x_cur = jnp.concatenate([c[pi] for c, _ in pairs], axis=0)
        x_src = jnp.concatenate([s[pi] for _, s in pairs], axis=0)
        out = _lpt(p_l, f_cur, f_src, x_cur, x_src, k_l)
        return out.reshape((len(pairs), B) + out.shape[1:])

    def lpt_levels(pairs):
        a1 = lpt_all(pairs, p['gat1'], 16, 0, 1)
        a2 = lpt_all(pairs, p['gat2'], 16, 2, 3)
        a3 = lpt_all(pairs, p['gat3'], 8, 4, 5)
        return a1, a2, a3

    pairs = []
    for t in range(T):
        prev = encs[t - 1] if t > 0 else encs[0]
        nxt = encs[t + 1] if t < T - 1 else encs[t]
        pairs += [(encs[t], prev), (encs[t], nxt)]
    a1, a2, a3 = lpt_levels(pairs)
    for t in range(T):
        st = _lstm3(p, st, (a1[2 * t], a1[2 * t + 1], a2[2 * t],
                            a2[2 * t + 1], a3[2 * t], a3[2 * t + 1]))

    def decode(st, e, fine_xyz):
        H1, _, H2, _, H3, _ = st
        x2 = _fp(p['fp32'], H3, e[5], H2, e[3], 8)
        x1 = _fp(p['fp21'], x2, e[3], H1, e[1], 16)
        return _fpns_cls(p['fp10'], p['cls'], x1, e[1], fine_xyz, 32)

    num_steps = 2
    pc_next = decode(st, encs[-1], frames[-1])
    preds = [pc_next]
    for _ in range(1, num_steps):
        e_new = encode(pc_next)
        b1, b2, b3 = lpt_levels([(e_new, encs[-1]), (e_new, e_new)])
        st = _lstm3(p, st, (b1[0], b1[1], b2[0], b2[1], b3[0], b3[1]))
        encs.append(e_new)
        pc_next = decode(st, e_new, pc_next)
        preds.append(pc_next)
    return jnp.stack(preds)


# speedup vs baseline: 1.1994x; 1.0018x over previous
import functools

import jax
import jax.numpy as jnp
import numpy as np
from jax import lax
from jax.experimental import pallas as pl
from jax.experimental.pallas import tpu as pltpu
from jax.experimental.pallas import tpu_sc as plsc

_F32 = jnp.float32
_BIG = 3.0e38
_PREC = jax.lax.Precision.DEFAULT


def _dot(a, b):
    return jax.lax.dot_general(a, b, (((1,), (0,)), ((), ())),
                               preferred_element_type=_F32, precision=_PREC)


def _first_min_onehot(D, lane_iota, n):
    mn = jnp.min(D, axis=1, keepdims=True)
    idx = jnp.min(jnp.where(D == mn, lane_iota, n), axis=1, keepdims=True)
    oh = (lane_iota == idx).astype(_F32)
    return oh, mn, idx


def _fps_kernel(xs_ref, ys_ref, zs_ref, cx_ref, cy_ref, cz_ref, *, n, m):
    xs, ys, zs = xs_ref[...], ys_ref[...], zs_ref[...]
    x0, y0, z0 = xs[:, 0:1], ys[:, 0:1], zs[:, 0:1]
    iota = jax.lax.broadcasted_iota(jnp.int32, (1, n), 1)
    lane_m = jax.lax.broadcasted_iota(jnp.int32, (1, m), 1)
    d0 = (xs - x0) ** 2 + (ys - y0) ** 2 + (zs - z0) ** 2
    cx_ref[...] = jnp.broadcast_to(x0, cx_ref.shape)
    cy_ref[...] = jnp.broadcast_to(y0, cy_ref.shape)
    cz_ref[...] = jnp.broadcast_to(z0, cz_ref.shape)

    def body(i, dists):
        mx = jnp.max(dists, axis=1, keepdims=True)
        sel = jnp.min(jnp.where(dists == mx, iota, n), axis=1, keepdims=True)
        mask = (iota == sel).astype(_F32)
        xc = jnp.sum(xs * mask, axis=1, keepdims=True)
        yc = jnp.sum(ys * mask, axis=1, keepdims=True)
        zc = jnp.sum(zs * mask, axis=1, keepdims=True)
        dnew = (xs - xc) ** 2 + (ys - yc) ** 2 + (zs - zc) ** 2
        hit = lane_m == i
        cx_ref[...] = jnp.where(hit, xc, cx_ref[...])
        cy_ref[...] = jnp.where(hit, yc, cy_ref[...])
        cz_ref[...] = jnp.where(hit, zc, cz_ref[...])
        return jnp.minimum(dists, dnew)

    jax.lax.fori_loop(1, m, body, d0)


def _knn_idx_kernel(cen_ref, xyzT_ref, i_out_ref, d_ref, *, n, m, k):
    cen = cen_ref[...]
    sxyzT = xyzT_ref[...]
    D0 = ((cen[:, 0:1] - sxyzT[0:1, :]) ** 2
          + (cen[:, 1:2] - sxyzT[1:2, :]) ** 2)
    d_ref[...] = D0 + (cen[:, 2:3] - sxyzT[2:3, :]) ** 2
    lane_mn = jax.lax.broadcasted_iota(jnp.int32, (m, n), 1)

    def body(j, _):
        D = d_ref[...]
        oh, _, idx = _first_min_onehot(D, lane_mn, n)
        d_ref[...] = D + oh * _BIG
        i_out_ref[j] = idx
        return 0

    jax.lax.fori_loop(0, k, body, 0)


def _sc_gather(table, idx, D):
    info = plsc.get_sparse_core_info()
    NW = info.num_cores * info.num_subcores
    B = idx.shape[0]
    b_per_w = B // NW
    mesh = plsc.VectorSubcoreMesh(core_axis_name="c", subcore_axis_name="s")

    ch = min(b_per_w, 512)
    n_ch = b_per_w // ch

    @functools.partial(
        pl.kernel, mesh=mesh,
        out_type=jax.ShapeDtypeStruct((B, D), jnp.float32),
        scratch_types=[
            pltpu.VMEM((ch,), jnp.int32),
            pltpu.VMEM((ch, D), jnp.float32),
            pltpu.SemaphoreType.DMA,
        ],
    )
    def gk(table_hbm, idx_hbm, out_hbm, idx_v, rows_v, sem):
        wid = lax.axis_index("s") * info.num_cores + lax.axis_index("c")
        base = wid * b_per_w
        for c in range(n_ch):
            off = base + c * ch
            pltpu.sync_copy(idx_hbm.at[pl.ds(off, ch)], idx_v)
            pltpu.async_copy(table_hbm.at[idx_v], rows_v, sem).wait()
            pltpu.sync_copy(rows_v, out_hbm.at[pl.ds(off, ch)])

    return gk(table, idx)


def _sa_mlp_kernel(rows_ref, cen_ref,
                   w1_ref, b1_ref, w2_ref, b2_ref, w3_ref, b3_ref,
                   fout_ref, *, m, k, cf, cpad):
    cen = cen_ref[...]
    cenpad = jnp.concatenate(
        [jnp.zeros((m, cf), _F32), cen,
         jnp.zeros((m, cpad - cf - 3), _F32)], axis=1)
    h = rows_ref[...].reshape(k, m, cpad) - cenpad[None, :, :]
    a = jnp.maximum(_dot(h.reshape(k * m, cpad), w1_ref[...]) + b1_ref[...], 0.0)
    a = jnp.maximum(_dot(a, w2_ref[...]) + b2_ref[...], 0.0)
    a = jnp.maximum(_dot(a, w3_ref[...]) + b3_ref[...], 0.0)
    fout_ref[...] = jnp.max(a.reshape(k, m, -1), axis=0)


def _sa_kernel(featxyz_ref, cen_ref, xyzT_ref,
               w1_ref, b1_ref, w2_ref, b2_ref, w3_ref, b3_ref,
               fout_ref, d_ref,
               *, n, m, k, cf):
    cen = cen_ref[...]
    sxyzT = xyzT_ref[...]
    D0 = ((cen[:, 0:1] - sxyzT[0:1, :]) ** 2
          + (cen[:, 1:2] - sxyzT[1:2, :]) ** 2)
    d_ref[...] = D0 + (cen[:, 2:3] - sxyzT[2:3, :]) ** 2

    lane_mn = jax.lax.broadcasted_iota(jnp.int32, (m, n), 1)
    cenpad = jnp.concatenate([jnp.zeros((m, cf), _F32), cen], axis=1)
    featxyz = featxyz_ref[...]
    w1, b1 = w1_ref[...], b1_ref[...]
    w2, b2 = w2_ref[...], b2_ref[...]
    w3, b3 = w3_ref[...], b3_ref[...]

    def nbr_body(_, acc):
        D = d_ref[...]
        oh, _, idx = _first_min_onehot(D, lane_mn, n)
        d_ref[...] = D + oh * _BIG
        g = _dot(oh, featxyz) - cenpad
        a = jnp.maximum(_dot(g, w1) + b1, 0.0)
        a = jnp.maximum(_dot(a, w2) + b2, 0.0)
        a = jnp.maximum(_dot(a, w3) + b3, 0.0)
        return jnp.maximum(acc, a)

    cout = w3.shape[1]
    fout_ref[...] = jax.lax.fori_loop(
        0, k, nbr_body, jnp.zeros((m, cout), _F32))


def _sa(layers, feat, xyz, m, k):
    G, n, cf = feat.shape
    (w1, b1), (w2, b2), (w3, b3) = layers
    cout = w3.shape[1]
    featxyz = jnp.concatenate([feat, xyz], axis=2)
    xyzT = jnp.transpose(xyz, (0, 2, 1))
    cx, cy, cz = pl.pallas_call(
        functools.partial(_fps_kernel, n=n, m=m),
        out_shape=(jax.ShapeDtypeStruct((G, m), _F32),) * 3,
    )(xyzT[:, 0], xyzT[:, 1], xyzT[:, 2])
    cen = jnp.stack([cx, cy, cz], axis=2)

    if n >= 2048:
        idx = jax.vmap(pl.pallas_call(
            functools.partial(_knn_idx_kernel, n=n, m=m, k=k),
            out_shape=jax.ShapeDtypeStruct((k, m, 1), jnp.int32),
            scratch_shapes=[pltpu.VMEM((m, n), _F32)],
        ), in_axes=(0, 0))(cen, xyzT)
        cpad = 128 * ((cf + 3 + 127) // 128)
        table = jnp.pad(featxyz, ((0, 0), (0, 0), (0, cpad - cf - 3)))
        idx_flat = (idx.reshape(G, k * m)
                    + (jnp.arange(G, dtype=jnp.int32) * n)[:, None]).reshape(-1)
        rows = _sc_gather(table.reshape(G * n, cpad), idx_flat, cpad)
        fn = pl.pallas_call(
            functools.partial(_sa_mlp_kernel, m=m, k=k, cf=cf, cpad=cpad),
            out_shape=jax.ShapeDtypeStruct((m, cout), _F32),
        )
        w1p = jnp.pad(w1, ((0, cpad - cf - 3), (0, 0)))
        fout = jax.vmap(fn, in_axes=(0, 0) + (None,) * 6)(
            rows.reshape(G, k * m, cpad), cen,
            w1p, b1.reshape(1, -1), w2, b2.reshape(1, -1),
            w3, b3.reshape(1, -1))
        return fout, cen

    fn = pl.pallas_call(
        functools.partial(_sa_kernel, n=n, m=m, k=k, cf=cf),
        out_shape=jax.ShapeDtypeStruct((m, cout), _F32),
        scratch_shapes=[pltpu.VMEM((m, n), _F32)],
    )
    fout = jax.vmap(fn, in_axes=(0, 0, 0) + (None,) * 6)(
        featxyz, cen, xyzT,
        w1, b1.reshape(1, -1), w2, b2.reshape(1, -1), w3, b3.reshape(1, -1))
    return fout, cen


def _lpt_kernel(fcur_ref, fsrc_ref, qxyz_ref, sxyzT_ref,
                wq_ref, wk_ref, wv_ref, out_ref, d_ref, s_ref, i_ref,
                *, m, n, k, c):
    qxyz = qxyz_ref[...]
    sxyzT = sxyzT_ref[...]
    D0 = ((qxyz[:, 0:1] - sxyzT[0:1, :]) ** 2
          + (qxyz[:, 1:2] - sxyzT[1:2, :]) ** 2)
    d_ref[...] = D0 + (qxyz[:, 2:3] - sxyzT[2:3, :]) ** 2
    lane_mn = jax.lax.broadcasted_iota(jnp.int32, (m, n), 1)

    q = _dot(fcur_ref[...], wq_ref[...])
    K = _dot(fsrc_ref[...], wk_ref[...])
    V = _dot(fsrc_ref[...], wv_ref[...])
    S = jax.lax.dot_general(q, K, (((1,), (1,)), ((), ())),
                            preferred_element_type=_F32,
                            precision=_PREC)

    def sel_body(j, _):
        D = d_ref[...]
        oh, _, idx = _first_min_onehot(D, lane_mn, n)
        d_ref[...] = D + oh * _BIG
        s_ref[j] = jnp.sum(S * oh, axis=1, keepdims=True)
        i_ref[j] = idx
        return 0

    jax.lax.fori_loop(0, k, sel_body, 0)

    s = s_ref[...] / np.sqrt(c)
    e = jnp.exp(s - jnp.max(s, axis=0, keepdims=True))
    s_ref[...] = e / jnp.sum(e, axis=0, keepdims=True)

    d_ref[...] = jnp.zeros((m, n), _F32)

    def mix_body(j, _):
        d_ref[...] = d_ref[...] + s_ref[j] * (lane_mn == i_ref[j]).astype(_F32)
        return 0

    jax.lax.fori_loop(0, k, mix_body, 0)
    out_ref[...] = _dot(d_ref[...], V)


def _lpt(p, f_cur, f_src, xyz_cur, xyz_src, k):
    B, m, c = f_cur.shape
    n = f_src.shape[1]
    sxyzT = jnp.transpose(xyz_src, (0, 2, 1))
    fn = pl.pallas_call(
        functools.partial(_lpt_kernel, m=m, n=n, k=k, c=c),
        out_shape=jax.ShapeDtypeStruct((m, c), _F32),
        scratch_shapes=[pltpu.VMEM((m, n), _F32),
                        pltpu.VMEM((k, m, 1), _F32),
                        pltpu.VMEM((k, m, 1), jnp.int32)],
    )
    return jax.vmap(fn, in_axes=(0, 0, 0, 0, None, None, None))(
        f_cur, f_src, xyz_cur, sxyzT, p['Wq'], p['Wk'], p['Wv'])


def _lstm3_kernel(*refs):
    for lvl in range(3):
        fb_ref, ff_ref, h_ref, c_ref = refs[4 * lvl:4 * lvl + 4]
        wx_ref, wh_ref, b_ref = refs[12 + 3 * lvl:15 + 3 * lvl]
        hout_ref, cout_ref = refs[21 + 2 * lvl:23 + 2 * lvl]
        hdim = h_ref.shape[1]
        x = jnp.concatenate([fb_ref[...], ff_ref[...]], axis=1)
        g = _dot(x, wx_ref[...]) + _dot(h_ref[...], wh_ref[...]) + b_ref[...]
        i = jax.nn.sigmoid(g[:, 0:hdim])
        f = jax.nn.sigmoid(g[:, hdim:2 * hdim])
        gg = jnp.tanh(g[:, 2 * hdim:3 * hdim])
        o = jax.nn.sigmoid(g[:, 3 * hdim:4 * hdim])
        cn = f * c_ref[...] + i * gg
        hout_ref[...] = o * jnp.tanh(cn)
        cout_ref[...] = cn


def _lstm3(p, st, fbff):
    H1, C1, H2, C2, H3, C3 = st
    fb1, ff1, fb2, ff2, fb3, ff3 = fbff
    B = H1.shape[0]
    shapes = tuple(jax.ShapeDtypeStruct(h.shape[1:], _F32)
                   for h in (H1, H1, H2, H2, H3, H3))
    fn = pl.pallas_call(_lstm3_kernel, out_shape=shapes)
    ws = []
    for name in ('lstm1', 'lstm2', 'lstm3'):
        ws += [p[name]['Wx'], p[name]['Wh'], p[name]['b'].reshape(1, -1)]
    return jax.vmap(fn, in_axes=(0,) * 12 + (None,) * 9)(
        fb1, ff1, H1, C1, fb2, ff2, H2, C2, fb3, ff3, H3, C3, *ws)


def _interp(xc, posf, poscT, d_ref, w_ref, m, n, k):
    D0 = ((posf[:, 0:1] - poscT[0:1, :]) ** 2
          + (posf[:, 1:2] - poscT[1:2, :]) ** 2)
    d_ref[...] = D0 + (posf[:, 2:3] - poscT[2:3, :]) ** 2
    w_ref[...] = jnp.zeros((m, n), _F32)
    lane_mn = jax.lax.broadcasted_iota(jnp.int32, (m, n), 1)

    def body(_, wsum):
        D = d_ref[...]
        oh, mn, idx = _first_min_onehot(D, lane_mn, n)
        d_ref[...] = D + oh * _BIG
        w = 1.0 / (mn + 1e-2)
        w_ref[...] = w_ref[...] + w * oh
        return wsum + w

    wsum = jax.lax.fori_loop(0, k, body, jnp.zeros((m, 1), _F32))
    return _dot(w_ref[...], xc) / wsum


def _fp_kernel(xc_ref, posf_ref, poscT_ref, xskip_ref,
               w1_ref, b1_ref, w2_ref, b2_ref, out_ref, d_ref, w_ref,
               *, m, n, k):
    interp = _interp(xc_ref[...], posf_ref[...], poscT_ref[...], d_ref, w_ref,
                     m, n, k)
    h = jnp.concatenate([interp, xskip_ref[...]], axis=1)
    h = jnp.maximum(_dot(h, w1_ref[...]) + b1_ref[...], 0.0)
    h = jnp.maximum(_dot(h, w2_ref[...]) + b2_ref[...], 0.0)
    out_ref[...] = h


def _fp(layers, x_c, pos_c, x_skip, pos_f, k):
    B, m, _ = pos_f.shape
    n = pos_c.shape[1]
    (w1, b1), (w2, b2) = layers
    poscT = jnp.transpose(pos_c, (0, 2, 1))
    fn = pl.pallas_call(
        functools.partial(_fp_kernel, m=m, n=n, k=k),
        out_shape=jax.ShapeDtypeStruct((m, w2.shape[1]), _F32),
        scratch_shapes=[pltpu.VMEM((m, n), _F32), pltpu.VMEM((m, n), _F32)],
    )
    return jax.vmap(fn, in_axes=(0, 0, 0, 0, None, None, None, None))(
        x_c, pos_f, poscT, x_skip, w1, b1.reshape(1, -1), w2, b2.reshape(1, -1))


def _fpns_cls_kernel(xc_ref, posf_ref, poscT_ref,
                     w1_ref, b1_ref, w2_ref, b2_ref,
                     c1_ref, c2_ref, c3_ref, c4_ref, out_ref, d_ref, w_ref,
                     *, m, n, k):
    interp = _interp(xc_ref[...], posf_ref[...], poscT_ref[...], d_ref, w_ref,
                     m, n, k)
    h = jnp.maximum(_dot(interp, w1_ref[...]) + b1_ref[...], 0.0)
    h = jnp.maximum(_dot(h, w2_ref[...]) + b2_ref[...], 0.0)
    h = _dot(h, c1_ref[...])
    h = _dot(h, c2_ref[...])
    h = _dot(h, c3_ref[...])
    h = _dot(h, c4_ref[...])
    out_ref[...] = posf_ref[...] + h


def _fpns_cls(layers, cls, x_c, pos_c, pos_f, k):
    B, m, _ = pos_f.shape
    n = pos_c.shape[1]
    (w1, b1), (w2, b2) = layers
    c1, c2, c3, c4 = cls
    poscT = jnp.transpose(pos_c, (0, 2, 1))
    fn = pl.pallas_call(
        functools.partial(_fpns_cls_kernel, m=m, n=n, k=k),
        out_shape=jax.ShapeDtypeStruct((m, 3), _F32),
        scratch_shapes=[pltpu.VMEM((m, n), _F32), pltpu.VMEM((m, n), _F32)],
    )
    return jax.vmap(fn, in_axes=(0, 0, 0) + (None,) * 8)(
        x_c, pos_f, poscT, w1, b1.reshape(1, -1), w2, b2.reshape(1, -1),
        c1, c2, c3, c4)


def kernel(input_xyz, num_pred, params):
    p = params
    T, B, _, N = input_xyz.shape
    frames = jnp.transpose(input_xyz, (0, 1, 3, 2))
    N1, N2, N3 = N // 16, N // 32, N // 64

    def encode(fr):
        f1, x1 = _sa(p['sa1'], fr, fr, N1, 32)
        f2, x2 = _sa(p['sa2'], f1, x1, N2, 16)
        f3, x3 = _sa(p['sa3'], f2, x2, N3, 8)
        return (f1, x1, f2, x2, f3, x3)

    e_all = encode(frames.reshape(T * B, N, 3))
    encs = [tuple(a.reshape((T, B) + a.shape[1:])[t] for a in e_all)
            for t in range(T)]

    st = (jnp.zeros((B, N1, 128), _F32), jnp.zeros((B, N1, 128), _F32),
          jnp.zeros((B, N2, 256), _F32), jnp.zeros((B, N2, 256), _F32),
          jnp.zeros((B, N3, 512), _F32), jnp.zeros((B, N3, 512), _F32))

    def lpt_all(pairs, p_l, k_l, fi, pi):
        f_cur = jnp.concatenate([c[fi] for c, _ in pairs], axis=0)
        f_src = jnp.concatenate([s[fi] for _, s in pairs], axis=0)
        x_cur = jnp.concatenate([c[pi] for c, _ in pairs], axis=0)
        x_src = jnp.concatenate([s[pi] for _, s in pairs], axis=0)
        out = _lpt(p_l, f_cur, f_src, x_cur, x_src, k_l)
        return out.reshape((len(pairs), B) + out.shape[1:])

    def lpt_levels(pairs):
        a1 = lpt_all(pairs, p['gat1'], 16, 0, 1)
        a2 = lpt_all(pairs, p['gat2'], 16, 2, 3)
        a3 = lpt_all(pairs, p['gat3'], 8, 4, 5)
        return a1, a2, a3

    pairs = []
    for t in range(T):
        prev = encs[t - 1] if t > 0 else encs[0]
        nxt = encs[t + 1] if t < T - 1 else encs[t]
        pairs += [(encs[t], prev), (encs[t], nxt)]
    a1, a2, a3 = lpt_levels(pairs)
    for t in range(T):
        st = _lstm3(p, st, (a1[2 * t], a1[2 * t + 1], a2[2 * t],
                            a2[2 * t + 1], a3[2 * t], a3[2 * t + 1]))

    def decode(st, e, fine_xyz):
        H1, _, H2, _, H3, _ = st
        x2 = _fp(p['fp32'], H3, e[5], H2, e[3], 8)
        x1 = _fp(p['fp21'], x2, e[3], H1, e[1], 16)
        return _fpns_cls(p['fp10'], p['cls'], x1, e[1], fine_xyz, 32)

    num_steps = 2
    pc_next = decode(st, encs[-1], frames[-1])
    preds = [pc_next]
    for _ in range(1, num_steps):
        e_new = encode(pc_next)
        b1, b2, b3 = lpt_levels([(e_new, encs[-1]), (e_new, e_new)])
        st = _lstm3(p, st, (b1[0], b1[1], b2[0], b2[1], b3[0], b3[1]))
        encs.append(e_new)
        pc_next = decode(st, e_new, pc_next)
        preds.append(pc_next)
    return jnp.stack(preds)
